# baseline (speedup 1.0000x reference)
import jax
import jax.numpy as jnp
from jax import lax
from jax.experimental import pallas as pl
from jax.experimental.pallas import tpu as pltpu
from jax.experimental.pallas import tpu_sc as plsc

VOCAB = 30522
D = 768
S = 512
B = 128
N = S * B
EPS = 1e-12

NC = 2
NS = 16
NW = NC * NS

K = 4
SC_CHUNK = S // K
NK = SC_CHUNK * B
TOK_PER_W = NK // NW
CHUNK = 64
NCHUNK = TOK_PER_W // CHUNK
NBUF = 2


def _sc_gather_body(table, idx_hbm, out_hbm, idx_v, rows0, rows1,
                    gsem0, gsem1, osem0, osem1):
    wid = lax.axis_index("s") * NC + lax.axis_index("c")
    base = wid * TOK_PER_W
    pltpu.sync_copy(idx_hbm.at[pl.ds(base, TOK_PER_W)], idx_v)

    bufs = (rows0, rows1)
    gsems = (gsem0, gsem1)
    osems = (osem0, osem1)

    def gather_desc(g, b):
        return pltpu.make_async_copy(
            table.at[idx_v.at[pl.ds(g * CHUNK, CHUNK)]], bufs[b], gsems[b])

    def out_desc(g, b):
        return pltpu.make_async_copy(
            bufs[b], out_hbm.at[pl.ds(base + g * CHUNK, CHUNK)], osems[b])

    def step(g2, _):
        for b in range(NBUF):
            g = g2 * NBUF + b

            @pl.when(g >= NBUF)
            def _():
                out_desc(g - NBUF, b).wait()

            d = gather_desc(g, b)
            d.start()
            d.wait()
            out_desc(g, b).start()
        return 0

    lax.fori_loop(0, NCHUNK // NBUF, step, 0)
    for b in range(NBUF):
        out_desc(NCHUNK - NBUF + b, b).wait()


def _sc_gather(word_emb, ids_chunk):
    mesh = plsc.VectorSubcoreMesh(core_axis_name="c", subcore_axis_name="s")
    fn = pl.kernel(
        _sc_gather_body,
        out_type=jax.ShapeDtypeStruct((NK, D), jnp.float32),
        mesh=mesh,
        scratch_types=[
            pltpu.VMEM((TOK_PER_W,), jnp.int32),
            pltpu.VMEM((CHUNK, D), jnp.float32),
            pltpu.VMEM((CHUNK, D), jnp.float32),
            pltpu.SemaphoreType.DMA,
            pltpu.SemaphoreType.DMA,
            pltpu.SemaphoreType.DMA,
            pltpu.SemaphoreType.DMA,
        ],
    )
    return fn(word_emb, ids_chunk)


SBLK = 16
STEPS_PER_K = SC_CHUNK // SBLK


def _ln_compute(g_ref, pe_ref, te_ref, gamma_ref, beta_ref, o_ref):
    emb = g_ref[...] + pe_ref[...][:, None, :] + te_ref[...][None, :, :]
    mean = jnp.mean(emb, axis=-1, keepdims=True)
    cen = emb - mean
    var = jnp.mean(cen * cen, axis=-1, keepdims=True)
    o_ref[...] = (cen * lax.rsqrt(var + EPS)) * gamma_ref[...] + beta_ref[...]


def _ln_body_first(g_ref, pe_ref, te_ref, gamma_ref, beta_ref, o_ref):
    _ln_compute(g_ref, pe_ref, te_ref, gamma_ref, beta_ref, o_ref)


def _ln_body_chain(g_ref, pe_ref, te_ref, gamma_ref, beta_ref, prev_ref,
                   o_ref):
    del prev_ref
    _ln_compute(g_ref, pe_ref, te_ref, gamma_ref, beta_ref, o_ref)


def _ln_chunk(k, gathered_k, pe_k, tok_row, ln_gamma, ln_beta, prev):
    base = k * STEPS_PER_K
    out_spec = pl.BlockSpec((SBLK, B, D), lambda i: (base + i, 0, 0))
    in_specs = [
        pl.BlockSpec((SBLK, B, D), lambda i: (i, 0, 0)),
        pl.BlockSpec((SBLK, D), lambda i: (i, 0)),
        pl.BlockSpec((1, D), lambda i: (0, 0)),
        pl.BlockSpec((1, D), lambda i: (0, 0)),
        pl.BlockSpec((1, D), lambda i: (0, 0)),
    ]
    args = [gathered_k, pe_k, tok_row, ln_gamma, ln_beta]
    if prev is None:
        body = _ln_body_first
        aliases = {}
    else:
        body = _ln_body_chain
        in_specs.append(pl.BlockSpec(memory_space=pl.ANY))
        args.append(prev)
        aliases = {5: 0}
    return pl.pallas_call(
        body,
        grid=(STEPS_PER_K,),
        in_specs=in_specs,
        out_specs=out_spec,
        out_shape=jax.ShapeDtypeStruct((S, B, D), jnp.float32),
        input_output_aliases=aliases,
    )(*args)


def kernel(input_ids, word_emb, pos_emb, tok_emb, ln_gamma, ln_beta):
    ids_flat = input_ids.reshape(N).astype(jnp.int32)
    tok_row = lax.slice(tok_emb, (0, 0), (1, D))
    gamma = ln_gamma.reshape(1, D)
    beta = ln_beta.reshape(1, D)

    gathered = [
        _sc_gather(word_emb, lax.slice(ids_flat, (k * NK,), ((k + 1) * NK,)))
        for k in range(K)
    ]
    out = None
    for k in range(K):
        pe_k = lax.slice(pos_emb, (k * SC_CHUNK, 0), ((k + 1) * SC_CHUNK, D))
        out = _ln_chunk(k, gathered[k].reshape(SC_CHUNK, B, D), pe_k,
                        tok_row, gamma, beta, out)
    return out


# speedup vs baseline: 1.0243x; 1.0243x over previous
import jax
import jax.numpy as jnp
from jax import lax
from jax.experimental import pallas as pl
from jax.experimental.pallas import tpu as pltpu
from jax.experimental.pallas import tpu_sc as plsc

VOCAB = 30522
D = 768
S = 512
B = 128
N = S * B
EPS = 1e-12

NC = 2
NS = 16
NW = NC * NS

K = 2
SC_CHUNK = S // K
NK = SC_CHUNK * B
TOK_PER_W = NK // NW
CHUNK = 64
NCHUNK = TOK_PER_W // CHUNK
NBUF = 2


def _sc_gather_body(table, idx_hbm, out_hbm, idx_v, rows0, rows1,
                    gsem0, gsem1, osem0, osem1):
    wid = lax.axis_index("s") * NC + lax.axis_index("c")
    base = wid * TOK_PER_W
    pltpu.sync_copy(idx_hbm.at[pl.ds(base, TOK_PER_W)], idx_v)

    bufs = (rows0, rows1)
    gsems = (gsem0, gsem1)
    osems = (osem0, osem1)

    def gather_desc(g, b):
        return pltpu.make_async_copy(
            table.at[idx_v.at[pl.ds(g * CHUNK, CHUNK)]], bufs[b], gsems[b])

    def out_desc(g, b):
        return pltpu.make_async_copy(
            bufs[b], out_hbm.at[pl.ds(base + g * CHUNK, CHUNK)], osems[b])

    def step(g2, _):
        for b in range(NBUF):
            g = g2 * NBUF + b

            @pl.when(g >= NBUF)
            def _():
                out_desc(g - NBUF, b).wait()

            d = gather_desc(g, b)
            d.start()
            d.wait()
            out_desc(g, b).start()
        return 0

    lax.fori_loop(0, NCHUNK // NBUF, step, 0)
    for b in range(NBUF):
        out_desc(NCHUNK - NBUF + b, b).wait()


def _sc_gather(word_emb, ids_chunk):
    mesh = plsc.VectorSubcoreMesh(core_axis_name="c", subcore_axis_name="s")
    fn = pl.kernel(
        _sc_gather_body,
        out_type=jax.ShapeDtypeStruct((NK, D), jnp.float32),
        mesh=mesh,
        scratch_types=[
            pltpu.VMEM((TOK_PER_W,), jnp.int32),
            pltpu.VMEM((CHUNK, D), jnp.float32),
            pltpu.VMEM((CHUNK, D), jnp.float32),
            pltpu.SemaphoreType.DMA,
            pltpu.SemaphoreType.DMA,
            pltpu.SemaphoreType.DMA,
            pltpu.SemaphoreType.DMA,
        ],
    )
    return fn(word_emb, ids_chunk)


SBLK = 16
STEPS_PER_K = SC_CHUNK // SBLK


def _ln_compute(g_ref, pe_ref, te_ref, gamma_ref, beta_ref, o_ref):
    emb = g_ref[...] + pe_ref[...][:, None, :] + te_ref[...][None, :, :]
    mean = jnp.mean(emb, axis=-1, keepdims=True)
    cen = emb - mean
    var = jnp.mean(cen * cen, axis=-1, keepdims=True)
    o_ref[...] = (cen * lax.rsqrt(var + EPS)) * gamma_ref[...] + beta_ref[...]


def _ln_body_first(g_ref, pe_ref, te_ref, gamma_ref, beta_ref, o_ref):
    _ln_compute(g_ref, pe_ref, te_ref, gamma_ref, beta_ref, o_ref)


def _ln_body_chain(g_ref, pe_ref, te_ref, gamma_ref, beta_ref, prev_ref,
                   o_ref):
    del prev_ref
    _ln_compute(g_ref, pe_ref, te_ref, gamma_ref, beta_ref, o_ref)


def _ln_chunk(k, gathered_k, pe_k, tok_row, ln_gamma, ln_beta, prev):
    base = k * STEPS_PER_K
    out_spec = pl.BlockSpec((SBLK, B, D), lambda i: (base + i, 0, 0))
    in_specs = [
        pl.BlockSpec((SBLK, B, D), lambda i: (i, 0, 0)),
        pl.BlockSpec((SBLK, D), lambda i: (i, 0)),
        pl.BlockSpec((1, D), lambda i: (0, 0)),
        pl.BlockSpec((1, D), lambda i: (0, 0)),
        pl.BlockSpec((1, D), lambda i: (0, 0)),
    ]
    args = [gathered_k, pe_k, tok_row, ln_gamma, ln_beta]
    if prev is None:
        body = _ln_body_first
        aliases = {}
    else:
        body = _ln_body_chain
        in_specs.append(pl.BlockSpec(memory_space=pl.ANY))
        args.append(prev)
        aliases = {5: 0}
    return pl.pallas_call(
        body,
        grid=(STEPS_PER_K,),
        in_specs=in_specs,
        out_specs=out_spec,
        out_shape=jax.ShapeDtypeStruct((S, B, D), jnp.float32),
        input_output_aliases=aliases,
    )(*args)


def kernel(input_ids, word_emb, pos_emb, tok_emb, ln_gamma, ln_beta):
    ids_flat = input_ids.reshape(N).astype(jnp.int32)
    tok_row = lax.slice(tok_emb, (0, 0), (1, D))
    gamma = ln_gamma.reshape(1, D)
    beta = ln_beta.reshape(1, D)

    gathered = [
        _sc_gather(word_emb, lax.slice(ids_flat, (k * NK,), ((k + 1) * NK,)))
        for k in range(K)
    ]
    out = None
    for k in range(K):
        pe_k = lax.slice(pos_emb, (k * SC_CHUNK, 0), ((k + 1) * SC_CHUNK, D))
        out = _ln_chunk(k, gathered[k].reshape(SC_CHUNK, B, D), pe_k,
                        tok_row, gamma, beta, out)
    return out
